# Initial kernel scaffold; baseline (speedup 1.0000x reference)
import functools
import jax
import jax.numpy as jnp
from jax import lax
from jax.experimental import pallas as pl
from jax.experimental.pallas import tpu as pltpu
from jax.experimental.pallas import tpu_sc as plsc

DIM = 64
NW = 32
GSZ = 128


def _sc_gather(idx3d, weight, total):
    ng = total // (NW * GSZ)
    per_w = ng * GSZ
    mesh = plsc.VectorSubcoreMesh(core_axis_name="c", subcore_axis_name="s")

    @functools.partial(
        pl.kernel,
        out_type=jax.ShapeDtypeStruct((total, DIM), jnp.float32),
        mesh=mesh,
        scratch_types=[
            pltpu.VMEM((ng, GSZ), jnp.int32),
            pltpu.VMEM((GSZ, DIM), jnp.float32),
            pltpu.SemaphoreType.DMA,
        ],
    )
    def k(table_hbm, idx_hbm, out_hbm, idx_v, rows_v, sem):
        wid = lax.axis_index("s") * 2 + lax.axis_index("c")
        pltpu.sync_copy(idx_hbm.at[wid], idx_v)
        base = wid * per_w

        def body(g, carry):
            pltpu.async_copy(table_hbm.at[idx_v.at[g]], rows_v, sem).wait()
            pltpu.sync_copy(rows_v, out_hbm.at[pl.ds(base + g * GSZ, GSZ)])
            return carry

        lax.fori_loop(0, ng, body, 0)

    return k(weight, idx3d)


def kernel(indices, weight):
    b, s = indices.shape
    total = b * s
    idx3d = indices.astype(jnp.int32).reshape(NW, total // (NW * GSZ), GSZ)
    out = _sc_gather(idx3d, weight, total)
    return out.reshape(b, s, DIM)


# speedup vs baseline: 1.6833x; 1.6833x over previous
import functools
import jax
import jax.numpy as jnp
from jax import lax
from jax.experimental import pallas as pl
from jax.experimental.pallas import tpu as pltpu
from jax.experimental.pallas import tpu_sc as plsc

DIM = 64
NW = 32
GSZ = 128


def _sc_gather(idx3d, weight, total):
    ng = total // (NW * GSZ)
    per_w = ng * GSZ
    mesh = plsc.VectorSubcoreMesh(core_axis_name="c", subcore_axis_name="s")

    @functools.partial(
        pl.kernel,
        out_type=jax.ShapeDtypeStruct((total, DIM), jnp.float32),
        mesh=mesh,
        scratch_types=[
            pltpu.VMEM((ng, GSZ), jnp.int32),
            pltpu.VMEM((GSZ, DIM), jnp.float32),
            pltpu.SemaphoreType.DMA,
        ],
        compiler_params=pltpu.CompilerParams(use_tc_tiling_on_sc=False),
    )
    def k(table_hbm, idx_hbm, out_hbm, idx_v, rows_v, sem):
        wid = lax.axis_index("s") * 2 + lax.axis_index("c")
        pltpu.sync_copy(idx_hbm.at[wid], idx_v)
        base = wid * per_w

        def body(g, carry):
            pltpu.async_copy(table_hbm.at[idx_v.at[g]], rows_v, sem).wait()
            pltpu.sync_copy(rows_v, out_hbm.at[pl.ds(base + g * GSZ, GSZ)])
            return carry

        lax.fori_loop(0, ng, body, 0)

    return k(weight, idx3d)


def kernel(indices, weight):
    b, s = indices.shape
    total = b * s
    idx3d = indices.astype(jnp.int32).reshape(NW, total // (NW * GSZ), GSZ)
    out = _sc_gather(idx3d, weight, total)
    return out.reshape(b, s, DIM)


# speedup vs baseline: 1.8715x; 1.1118x over previous
import functools
import jax
import jax.numpy as jnp
from jax import lax
from jax.experimental import pallas as pl
from jax.experimental.pallas import tpu as pltpu
from jax.experimental.pallas import tpu_sc as plsc

DIM = 64
NW = 32
GSZ = 128


NBUF = 8


def _sc_gather(idx3d, weight, total):
    ng = total // (NW * GSZ)
    nblk = ng // NBUF
    per_w = ng * GSZ
    mesh = plsc.VectorSubcoreMesh(core_axis_name="c", subcore_axis_name="s")

    @functools.partial(
        pl.kernel,
        out_type=jax.ShapeDtypeStruct((total, DIM), jnp.float32),
        mesh=mesh,
        scratch_types=[
            pltpu.VMEM((ng, GSZ), jnp.int32),
            pltpu.VMEM((NBUF, GSZ, DIM), jnp.float32),
            pltpu.SemaphoreType.DMA((NBUF,)),
            pltpu.SemaphoreType.DMA((NBUF,)),
        ],
        compiler_params=pltpu.CompilerParams(use_tc_tiling_on_sc=False),
    )
    def k(table_hbm, idx_hbm, out_hbm, idx_v, rows_v, gsem, osem):
        wid = lax.axis_index("s") * 2 + lax.axis_index("c")
        pltpu.sync_copy(idx_hbm.at[wid], idx_v)
        base = wid * per_w

        def gather(g, b):
            pltpu.async_copy(table_hbm.at[idx_v.at[g]], rows_v.at[b], gsem.at[b])

        def write(g, b):
            pltpu.async_copy(
                rows_v.at[b], out_hbm.at[pl.ds(base + g * GSZ, GSZ)], osem.at[b]
            )

        def wait(sem, b, shape):
            pltpu.make_async_copy(
                table_hbm.at[pl.ds(0, GSZ)], rows_v.at[b], sem.at[b]
            ).wait()

        for b in range(NBUF):
            gather(b, b)

        def blk(G, carry):
            g0 = G * NBUF
            for b in range(NBUF):
                wait(gsem, b, None)
                write(g0 + b, b)
            for b in range(NBUF):
                wait(osem, b, None)
                gather(g0 + NBUF + b, b)
            return carry

        lax.fori_loop(0, nblk - 1, blk, 0)

        g0 = (nblk - 1) * NBUF
        for b in range(NBUF):
            wait(gsem, b, None)
            write(g0 + b, b)
        for b in range(NBUF):
            wait(osem, b, None)

    return k(weight, idx3d)


def kernel(indices, weight):
    b, s = indices.shape
    total = b * s
    idx3d = indices.astype(jnp.int32).reshape(NW, total // (NW * GSZ), GSZ)
    out = _sc_gather(idx3d, weight, total)
    return out.reshape(b, s, DIM)


# speedup vs baseline: 1.8765x; 1.0027x over previous
import functools
import jax
import jax.numpy as jnp
from jax import lax
from jax.experimental import pallas as pl
from jax.experimental.pallas import tpu as pltpu
from jax.experimental.pallas import tpu_sc as plsc

DIM = 64
NW = 32
GSZ = 128
NBUF = 8


def _sc_gather(idx_t, weight):
    s_len, b_len = idx_t.shape
    b_per_w = b_len // NW
    jblk = b_per_w // GSZ
    ng = s_len * jblk
    nblk = ng // NBUF
    mesh = plsc.VectorSubcoreMesh(core_axis_name="c", subcore_axis_name="s")

    @functools.partial(
        pl.kernel,
        out_type=jax.ShapeDtypeStruct((b_len, s_len, DIM), jnp.float32),
        mesh=mesh,
        scratch_types=[
            pltpu.VMEM((s_len, b_per_w), jnp.int32),
            pltpu.VMEM((NBUF, GSZ, DIM), jnp.float32),
            pltpu.SemaphoreType.DMA((NBUF,)),
            pltpu.SemaphoreType.DMA((NBUF,)),
        ],
        compiler_params=pltpu.CompilerParams(use_tc_tiling_on_sc=False),
    )
    def k(table_hbm, idx_hbm, out_hbm, idx_v, rows_v, gsem, osem):
        wid = lax.axis_index("s") * 2 + lax.axis_index("c")
        pltpu.sync_copy(idx_hbm.at[:, pl.ds(wid * b_per_w, b_per_w)], idx_v)
        base_b = wid * b_per_w

        def gather(g, b):
            j = g // s_len
            s = g - j * s_len
            pltpu.async_copy(
                table_hbm.at[idx_v.at[s, pl.ds(j * GSZ, GSZ)]],
                rows_v.at[b],
                gsem.at[b],
            )

        def write(g, b):
            j = g // s_len
            s = g - j * s_len
            pltpu.async_copy(
                rows_v.at[b],
                out_hbm.at[pl.ds(base_b + j * GSZ, GSZ), s],
                osem.at[b],
            )

        def wait(sem, b):
            pltpu.make_async_copy(
                table_hbm.at[pl.ds(0, GSZ)], rows_v.at[b], sem.at[b]
            ).wait()

        for b in range(NBUF):
            gather(b, b)

        def blk(G, carry):
            g0 = G * NBUF
            for b in range(NBUF):
                wait(gsem, b)
                write(g0 + b, b)
            for b in range(NBUF):
                wait(osem, b)
                gather(g0 + NBUF + b, b)
            return carry

        lax.fori_loop(0, nblk - 1, blk, 0)

        g0 = (nblk - 1) * NBUF
        for b in range(NBUF):
            wait(gsem, b)
            write(g0 + b, b)
        for b in range(NBUF):
            wait(osem, b)

    return k(weight, idx_t)


def kernel(indices, weight):
    idx_t = jnp.swapaxes(indices.astype(jnp.int32), 0, 1)
    return _sc_gather(idx_t, weight)


# speedup vs baseline: 1.8778x; 1.0007x over previous
import functools
import jax
import jax.numpy as jnp
from jax import lax
from jax.experimental import pallas as pl
from jax.experimental.pallas import tpu as pltpu
from jax.experimental.pallas import tpu_sc as plsc

DIM = 64
NW = 32
GSZ = 128
NBUF = 8


def _sc_gather(idx_t, weight):
    s_len, b_len = idx_t.shape
    b_per_w = b_len // NW
    jblk = b_per_w // GSZ
    ng = s_len * jblk
    nblk = ng // NBUF
    mesh = plsc.VectorSubcoreMesh(core_axis_name="c", subcore_axis_name="s")

    @functools.partial(
        pl.kernel,
        out_type=jax.ShapeDtypeStruct((b_len, s_len, DIM), jnp.float32),
        mesh=mesh,
        scratch_types=[
            pltpu.VMEM((s_len, b_per_w), jnp.int32),
            pltpu.VMEM((NBUF, GSZ, DIM), jnp.float32),
            pltpu.SemaphoreType.DMA((NBUF,)),
            pltpu.SemaphoreType.DMA((NBUF,)),
        ],
        compiler_params=pltpu.CompilerParams(use_tc_tiling_on_sc=False),
    )
    def k(table_hbm, idx_hbm, out_hbm, idx_v, rows_v, gsem, osem):
        wid = lax.axis_index("s") * 2 + lax.axis_index("c")
        pltpu.sync_copy(idx_hbm.at[:, pl.ds(wid * b_per_w, b_per_w)], idx_v)
        base_b = wid * b_per_w

        def gather(g, b):
            j = g // s_len
            s = g - j * s_len
            pltpu.async_copy(
                table_hbm.at[idx_v.at[s, pl.ds(j * GSZ, GSZ)]],
                rows_v.at[b],
                gsem.at[b],
            )

        def write(g, b):
            j = g // s_len
            s = g - j * s_len
            pltpu.async_copy(
                rows_v.at[b],
                out_hbm.at[pl.ds(base_b + j * GSZ, GSZ), s],
                osem.at[b],
            )

        def wait(sem, b):
            pltpu.make_async_copy(
                table_hbm.at[pl.ds(0, GSZ)], rows_v.at[b], sem.at[b]
            ).wait()

        for b in range(NBUF):
            gather(b, b)

        def blk(G, carry):
            g0 = G * NBUF
            for b in range(NBUF):
                wait(gsem, b)
                write(g0 + b, b)
            for b in range(NBUF):
                wait(osem, b)
                gather(g0 + NBUF + b, b)
            return carry

        lax.fori_loop(0, nblk - 1, blk, 0)

        g0 = (nblk - 1) * NBUF
        for b in range(NBUF):
            wait(gsem, b)
            write(g0 + b, b)
        for b in range(NBUF):
            wait(osem, b)

    return k(weight, idx_t)


def kernel(indices, weight):
    idx_t = jnp.swapaxes(indices.astype(jnp.int32), 0, 1)
    w128 = jax.lax.optimization_barrier(weight.reshape(500000, 128))
    return _sc_gather(idx_t, w128.reshape(1000000, DIM))


# speedup vs baseline: 2.5346x; 1.3498x over previous
import functools
import jax
import jax.numpy as jnp
from jax import lax
from jax.experimental import pallas as pl
from jax.experimental.pallas import tpu as pltpu
from jax.experimental.pallas import tpu_sc as plsc

DIM = 64
NW = 32
GSZ = 128
NBUF = 8


def _sc_gather(idx_t, weight):
    s_len, b_len = idx_t.shape
    b_per_w = b_len // NW
    jblk = b_per_w // GSZ
    ng = s_len * jblk
    nblk = ng // NBUF
    mesh = plsc.VectorSubcoreMesh(core_axis_name="c", subcore_axis_name="s")

    @functools.partial(
        pl.kernel,
        out_type=jax.ShapeDtypeStruct((b_len, 56, 128), jnp.float32),
        mesh=mesh,
        scratch_types=[
            pltpu.VMEM((s_len, b_per_w), jnp.int32),
            pltpu.VMEM((NBUF, GSZ, DIM), jnp.float32),
            pltpu.SemaphoreType.DMA((NBUF,)),
            pltpu.SemaphoreType.DMA((NBUF,)),
        ],
        compiler_params=pltpu.CompilerParams(use_tc_tiling_on_sc=False),
    )
    def k(table_hbm, idx_hbm, out_hbm, idx_v, rows_v, gsem, osem):
        wid = lax.axis_index("s") * 2 + lax.axis_index("c")
        pltpu.sync_copy(idx_hbm.at[:, pl.ds(wid * b_per_w, b_per_w)], idx_v)
        base_b = wid * b_per_w

        def gather(g, b):
            j = g // s_len
            s = g - j * s_len
            pltpu.async_copy(
                table_hbm.at[idx_v.at[s, pl.ds(j * GSZ, GSZ)]],
                rows_v.at[b],
                gsem.at[b],
            )

        def write(g, b):
            j = g // s_len
            s = g - j * s_len
            pltpu.async_copy(
                rows_v.at[b],
                out_hbm.at[pl.ds(base_b + j * GSZ, GSZ), s, pl.ds(0, DIM)],
                osem.at[b],
            )

        def wait(sem, b):
            pltpu.make_async_copy(
                table_hbm.at[pl.ds(0, GSZ)], rows_v.at[b], sem.at[b]
            ).wait()

        for b in range(NBUF):
            gather(b, b)

        def blk(G, carry):
            g0 = G * NBUF
            for b in range(NBUF):
                wait(gsem, b)
                write(g0 + b, b)
            for b in range(NBUF):
                wait(osem, b)
                gather(g0 + NBUF + b, b)
            return carry

        lax.fori_loop(0, nblk - 1, blk, 0)

        g0 = (nblk - 1) * NBUF
        for b in range(NBUF):
            wait(gsem, b)
            write(g0 + b, b)
        for b in range(NBUF):
            wait(osem, b)

    return k(weight, idx_t)


def kernel(indices, weight):
    idx_t = jnp.swapaxes(indices.astype(jnp.int32), 0, 1)
    out_p = _sc_gather(idx_t, weight)
    return out_p[:, :indices.shape[1], :DIM]
